# baseline (speedup 1.0000x reference)
import jax
import jax.numpy as jnp
from jax import lax
from jax.experimental import pallas as pl
from jax.experimental.pallas import tpu as pltpu
from jax.experimental.pallas import tpu_sc as plsc

B = 1024
D = 128
NW = 32
NB = B // NW
CH = 8
BLK = 256
VPAD = 32

PL_VOCABS = (21, 21, 13, 21, 4, 21, 21, 21, 21, 21, 21, 7)
CS_VOCABS = (21, 21, 13, 21, 4, 21)


def _sc_body(ps_i, tn_i, tc_i, ag_i, ps_T, tn_T, tc_T, ag_T, out,
             idx16a, idx16b, idx16c, idx80,
             rsmA, rsmB, rbigA, rbigB, t0v, outv0, outv1,
             semA, semB, semC, semD, semI, semF, semG):
    cid = lax.axis_index("c")
    sid = lax.axis_index("s")
    wid = sid * 2 + cid
    base = wid * NB

    pltpu.async_copy(ps_i.at[pl.ds(base * 16, NB * 16)], idx16a, semI)
    pltpu.async_copy(tn_i.at[pl.ds(base * 80, NB * 80)], idx80, semI)
    pltpu.async_copy(tc_i.at[pl.ds(base * 16, NB * 16)], idx16b, semI)
    pltpu.async_copy(ag_i.at[pl.ds(base * 16, NB * 16)], idx16c, semI)
    pltpu.make_async_copy(ps_i.at[pl.ds(0, NB * 16)], idx16a, semI).wait()
    pltpu.make_async_copy(tn_i.at[pl.ds(0, NB * 80)], idx80, semI).wait()
    pltpu.make_async_copy(tc_i.at[pl.ds(0, NB * 16)], idx16b, semI).wait()
    pltpu.make_async_copy(ag_i.at[pl.ds(0, NB * 16)], idx16c, semI).wait()

    zero16 = jnp.zeros((16,), jnp.float32)

    def ring_loop(n, fire, wait, consume):
        def g_body(g, _):
            c0 = 2 * g
            wait(0)
            consume(c0, 0)

            @pl.when(c0 + 2 < n)
            def _():
                fire(c0 + 2, 0)

            wait(1)
            consume(c0 + 1, 1)

            @pl.when(c0 + 3 < n)
            def _():
                fire(c0 + 3, 1)

            return 0

        lax.fori_loop(0, n // 2, g_body, 0)

    def store_row(outv, b, vecs, scale):
        for c in range(len(vecs)):
            outv[b, pl.ds(c * 16, 16)] = vecs[c] * scale

    def flush(outv, f, sem):
        pltpu.async_copy(outv, out.at[f, pl.ds(base, NB)], sem)

    def drain_flush(sem):
        pltpu.make_async_copy(out.at[0, pl.ds(0, NB)], outv0, sem).wait()

    def make_text(idx16, T_hbm, outv, masked):
        S = 16
        bufs = (rsmA, rsmB)
        sems = (semA, semB)

        def fire(c, k):
            pltpu.async_copy(T_hbm.at[idx16.at[pl.ds(c * CH * S, CH * S)]],
                             bufs[k], sems[k])

        def wait(k):
            pltpu.make_async_copy(T_hbm.at[pl.ds(0, CH * S)],
                                  bufs[k], sems[k]).wait()

        def consume(c, k):
            buf = bufs[k]

            def bb_body(bb, _):
                b = c * CH + bb

                def s_body(s, acc):
                    return tuple(acc[j] + buf[bb * S + s, pl.ds(j * 16, 16)]
                                 for j in range(8))

                acc = lax.fori_loop(0, S, s_body, (zero16,) * 8, unroll=4)
                if masked:
                    va = idx16[pl.ds(b * S, 16)]
                    nz = va[0] * 0
                    for s in range(16):
                        nz = nz + jnp.where(va[s] == 0, 1, 0)
                    nzv = lax.broadcast_in_dim(nz.astype(jnp.float32), (16,), ())
                    inv = 1.0 / jnp.maximum(jnp.float32(S) - nzv, 1.0)
                    vecs = tuple((acc[j] - nzv * t0v[0, pl.ds(j * 16, 16)]) * inv
                                 for j in range(8))
                    store_row(outv, b, vecs, 1.0)
                else:
                    store_row(outv, b, acc, 1.0 / S)
                return 0

            lax.fori_loop(0, CH, bb_body, 0)

        return fire, wait, consume

    tn_bufs = (rbigA, rbigB)
    tn_sems = (semC, semD)

    def tn_fire(b, k):
        pltpu.async_copy(tn_T.at[idx80.at[pl.ds(b * 80, 80)]],
                         tn_bufs[k], tn_sems[k])

    def tn_wait(k):
        pltpu.make_async_copy(tn_T.at[pl.ds(0, 80)],
                              tn_bufs[k], tn_sems[k]).wait()

    def tn_consume(b, k):
        buf = tn_bufs[k]

        def s_body(s, acc):
            return tuple(acc[j] + buf[s, pl.ds(j * 16, 16)]
                         + buf[s, pl.ds(128 + j * 16, 16)] for j in range(8))

        folded = lax.fori_loop(0, 80, s_body, (zero16,) * 8, unroll=4)
        store_row(outv1, b, folded, 1.0 / 160.0)

    ps_fns = make_text(idx16a, ps_T, outv0, False)
    tc_fns = make_text(idx16b, tc_T, outv0, False)
    ag_fns = make_text(idx16c, ag_T, outv1, True)

    tn_fire(0, 0)
    tn_fire(1, 1)

    ps_fns[0](0, 0)
    ps_fns[0](1, 1)
    ring_loop(NB // CH, *ps_fns)
    flush(outv0, 0, semF)

    tc_fns[0](0, 0)
    tc_fns[0](1, 1)

    ring_loop(NB, tn_fire, tn_wait, tn_consume)
    flush(outv1, 1, semG)
    pltpu.sync_copy(ag_T.at[pl.ds(0, 1)], t0v)

    drain_flush(semF)
    ring_loop(NB // CH, *tc_fns)
    flush(outv0, 2, semF)

    ag_fns[0](0, 0)
    ag_fns[0](1, 1)
    drain_flush(semG)
    ring_loop(NB // CH, *ag_fns)
    flush(outv1, 3, semG)

    drain_flush(semF)
    drain_flush(semG)


def _sc_forward(ps_i, tn_i, tc_i, ag_i, ps_T, tn_T, tc_T, ag_T):
    mesh = plsc.VectorSubcoreMesh(core_axis_name="c", subcore_axis_name="s",
                                  num_cores=2, num_subcores=16)
    scratch = [
        pltpu.VMEM((NB * 16,), jnp.int32),
        pltpu.VMEM((NB * 16,), jnp.int32),
        pltpu.VMEM((NB * 16,), jnp.int32),
        pltpu.VMEM((NB * 80,), jnp.int32),
        pltpu.VMEM((CH * 16, 128), jnp.float32),
        pltpu.VMEM((CH * 16, 128), jnp.float32),
        pltpu.VMEM((80, 256), jnp.float32),
        pltpu.VMEM((80, 256), jnp.float32),
        pltpu.VMEM((1, 128), jnp.float32),
        pltpu.VMEM((NB, 128), jnp.float32),
        pltpu.VMEM((NB, 128), jnp.float32),
        pltpu.SemaphoreType.DMA,
        pltpu.SemaphoreType.DMA,
        pltpu.SemaphoreType.DMA,
        pltpu.SemaphoreType.DMA,
        pltpu.SemaphoreType.DMA,
        pltpu.SemaphoreType.DMA,
        pltpu.SemaphoreType.DMA,
    ]
    fn = pl.kernel(_sc_body,
                   out_type=jax.ShapeDtypeStruct((4, B, D), jnp.float32),
                   mesh=mesh, scratch_types=scratch)
    return fn(ps_i, tn_i, tc_i, ag_i, ps_T, tn_T, tc_T, ag_T)


def _tc_body(*refs):
    sc_ref = refs[0]
    idxT_ref = refs[1]
    tbl_refs = refs[2:20]
    o = refs[20]

    o[:, 0:128] = sc_ref[0]
    o[:, 128:256] = sc_ref[1]
    o[:, 1792:1920] = sc_ref[2]
    o[:, 1920:2048] = sc_ref[3]

    iotaV = lax.broadcasted_iota(jnp.int32, (VPAD, BLK), 0)

    for f in range(12):
        v = PL_VOCABS[f]
        rows = idxT_ref[pl.ds(f * 20, 20), :]
        cnt = jnp.zeros((VPAD, BLK), jnp.float32)
        for s in range(20):
            cnt = cnt + (rows[s:s + 1, :] == iotaV).astype(jnp.float32)
        mm = lax.dot_general(cnt[0:v, :], tbl_refs[f][...],
                             (((0,), (0,)), ((), ())),
                             preferred_element_type=jnp.float32)
        o[:, 256 + f * 128:256 + (f + 1) * 128] = mm * (1.0 / 20.0)

    for j in range(6):
        v = CS_VOCABS[j]
        oh = (idxT_ref[pl.ds(240 + j, 1), :] == iotaV).astype(jnp.float32)
        mm = lax.dot_general(oh[0:v, :], tbl_refs[12 + j][...],
                             (((0,), (0,)), ((), ())),
                             preferred_element_type=jnp.float32)
        o[:, 2048 + j * 128:2048 + (j + 1) * 128] = mm


def _tc_small(sc_out, idxT, tbls):
    grid = (B // BLK,)
    in_specs = (
        [pl.BlockSpec((4, BLK, D), lambda i: (0, i, 0)),
         pl.BlockSpec((248, BLK), lambda i: (0, i))]
        + [pl.BlockSpec(t.shape, lambda i: (0, 0)) for t in tbls]
    )
    return pl.pallas_call(
        _tc_body,
        grid=grid,
        in_specs=in_specs,
        out_specs=pl.BlockSpec((BLK, 22 * D), lambda i: (i, 0)),
        out_shape=jax.ShapeDtypeStruct((B, 22 * D), jnp.float32),
    )(sc_out, idxT, *tbls)


def kernel(pl_name_src, track_name_pl, track_danceability_pl, track_energy_pl,
           track_key_pl, track_loudness_pl, track_mode_pl, track_speechiness_pl,
           track_acousticness_pl, track_instrumentalness_pl, track_liveness_pl,
           track_valence_pl, track_tempo_pl, time_signature_pl, track_name_can,
           artist_genres_can, track_danceability_can, track_energy_can,
           track_key_can, track_loudness_can, track_mode_can, track_speechiness_can,
           T_pl_name_src, T_track_name_pl, T_track_danceability_pl, T_track_energy_pl,
           T_track_key_pl, T_track_loudness_pl, T_track_mode_pl, T_track_speechiness_pl,
           T_track_acousticness_pl, T_track_instrumentalness_pl, T_track_liveness_pl,
           T_track_valence_pl, T_track_tempo_pl, T_time_signature_pl, T_track_name_can,
           T_artist_genres_can, T_track_danceability_can, T_track_energy_can,
           T_track_key_can, T_track_loudness_can, T_track_mode_can, T_track_speechiness_can):
    pl_idx = [track_danceability_pl, track_energy_pl, track_key_pl,
              track_loudness_pl, track_mode_pl, track_speechiness_pl,
              track_acousticness_pl, track_instrumentalness_pl,
              track_liveness_pl, track_valence_pl, track_tempo_pl,
              time_signature_pl]
    cs_idx = [track_danceability_can, track_energy_can, track_key_can,
              track_loudness_can, track_mode_can, track_speechiness_can]
    pl_Ts = [T_track_danceability_pl, T_track_energy_pl, T_track_key_pl,
             T_track_loudness_pl, T_track_mode_pl, T_track_speechiness_pl,
             T_track_acousticness_pl, T_track_instrumentalness_pl,
             T_track_liveness_pl, T_track_valence_pl, T_track_tempo_pl,
             T_time_signature_pl]
    cs_Ts = [T_track_danceability_can, T_track_energy_can, T_track_key_can,
             T_track_loudness_can, T_track_mode_can, T_track_speechiness_can]

    sc_out = _sc_forward(jnp.ravel(pl_name_src), jnp.ravel(track_name_pl),
                         jnp.ravel(track_name_can), jnp.ravel(artist_genres_can),
                         T_pl_name_src, T_track_name_pl,
                         T_track_name_can, T_artist_genres_can)

    idxT = jnp.concatenate(
        pl_idx + [x[:, None] for x in cs_idx]
        + [cs_idx[0][:, None], cs_idx[0][:, None]], axis=1).T
    return _tc_small(sc_out, idxT, pl_Ts + cs_Ts)
